# baseline (speedup 1.0000x reference)
import jax
import jax.numpy as jnp
from jax import lax
from jax.experimental import pallas as pl
from jax.experimental.pallas import tpu as pltpu
from jax.experimental.pallas import tpu_sc as plsc

_B, _S, _D = 4, 8192, 2048
_K = 64
_L = 16
_PCAP = 64
_CHECK = 32
_NREG = 4
_RC = _PCAP // _NREG
_FB = 128
_SB = _FB // _L
_CH = 128
_NCH = _S // _CH
_NWIN = _CH // _CHECK
_NC, _NS = 2, 16
_NW = _NC * _NS
_DBLK = _D // _FB
_GROUPS = _B * _DBLK
_GPW = _GROUPS // _NW
_NEG = float("-inf")


def _ce(vals, i, j):
    a, b = vals[i], vals[j]
    vals[i] = jnp.maximum(a, b)
    vals[j] = jnp.minimum(a, b)


def _sort_desc(vals):
    n = len(vals)
    k = 2
    while k <= n:
        j = k // 2
        while j >= 1:
            for i in range(n):
                l = i ^ j
                if l > i:
                    if (i & k) == 0:
                        _ce(vals, i, l)
                    else:
                        _ce(vals, l, i)
            j //= 2
        k *= 2


def _clean_desc(vals):
    n = len(vals)
    j = n // 2
    while j >= 1:
        for i in range(n):
            l = i ^ j
            if l > i:
                _ce(vals, i, l)
        j //= 2
    return vals


def _kmax_body(x_hbm, out_hbm, buf0, buf1, pend, top, obuf, cbuf, sem0, sem1):
    cid = lax.axis_index("c")
    sid = lax.axis_index("s")
    wid = sid * _NC + cid
    lanes = lax.iota(jnp.int32, _L)
    neg = jnp.full((_L,), _NEG, jnp.float32)

    def group_body(g, carry):
        gid = wid * _GPW + g
        b = gid // _DBLK
        d0 = (gid % _DBLK) * _FB

        def init_body(i, carry):
            top[i] = neg
            pend[pl.ds(i * _L, _L)] = neg
            return carry

        lax.fori_loop(0, _SB * _K, init_body, 0)

        def cinit_body(sb, carry):
            pb = lanes + sb * (_PCAP * _L)
            for r in range(_NREG):
                cbuf[sb * _NREG + r] = pb + r * (_RC * _L)
            return carry

        lax.fori_loop(0, _SB, cinit_body, 0)

        def dma(ci, bufp, semp):
            return pltpu.make_async_copy(
                x_hbm.at[b, pl.ds(ci * _CH, _CH), pl.ds(d0, _FB)], bufp, semp
            )

        dma(0, buf0, sem0).start()
        dma(1, buf1, sem1).start()

        def chunk_steps(ci, buf, semp):
            dma(ci, buf, semp).wait()

            def sb_body(sb, carry):
                base = sb * _K
                pbase = sb * (_PCAP * _L)
                rbases = [
                    lanes + (pbase + r * (_RC * _L)) for r in range(_NREG)
                ]

                def flush():
                    for h in range(2):
                        hb = pbase + h * (32 * _L)
                        p = [pend[pl.ds(hb + i * _L, _L)] for i in range(32)]
                        _sort_desc(p)
                        u = []
                        for i in range(32):
                            a = top[base + i]
                            bb = jnp.maximum(top[base + 32 + i], p[31 - i])
                            u.append(jnp.maximum(a, bb))
                            pend[pl.ds(hb + i * _L, _L)] = jnp.minimum(a, bb)
                        u = _clean_desc(u)
                        for i in range(32):
                            top[base + i] = u[i]
                        lo = [
                            pend[pl.ds(hb + i * _L, _L)] for i in range(32)
                        ]
                        lo = _clean_desc(lo)
                        for i in range(32):
                            top[base + 32 + i] = lo[i]
                    for i in range(_PCAP):
                        pend[pl.ds(pbase + i * _L, _L)] = neg

                def win_body(w, ct):
                    ptrs0, t = ct

                    @plsc.parallel_loop(
                        w * _CHECK, (w + 1) * _CHECK, step=_NREG,
                        unroll=2, carry=tuple(ptrs0),
                    )
                    def rows(ri, ps):
                        out = []
                        for r in range(_NREG):
                            v = buf[ri + r, pl.ds(sb * _L, _L)]
                            m = v > t
                            plsc.store_scatter(pend, [ps[r]], v, mask=m)
                            out.append(ps[r] + jnp.where(m, _L, 0))
                        return tuple(out)

                    ptrs = list(rows)
                    cm = ptrs[0] - rbases[0]
                    for r in range(1, _NREG):
                        cm = jnp.maximum(cm, ptrs[r] - rbases[r])
                    last = (ci == _NCH - 1) & (w == _NWIN - 1)
                    do_flush = jnp.any(cm > (_RC * _L) // 2) | last
                    pl.when(do_flush)(flush)
                    t = top[base + _K - 1]
                    ptrs = [
                        jnp.where(do_flush, rbases[r], ptrs[r])
                        for r in range(_NREG)
                    ]
                    return tuple(ptrs), t

                p0 = tuple(cbuf[sb * _NREG + r] for r in range(_NREG))
                t0 = top[base + _K - 1]
                p1, _t1 = lax.fori_loop(0, _NWIN, win_body, (p0, t0))
                for r in range(_NREG):
                    cbuf[sb * _NREG + r] = p1[r]
                return carry

            lax.fori_loop(0, _SB, sb_body, 0)

            def refill():
                dma(ci + 2, buf, semp).start()

            pl.when(ci + 2 < _NCH)(refill)

        def pair_body(cp, carry):
            chunk_steps(cp * 2, buf0, sem0)
            chunk_steps(cp * 2 + 1, buf1, sem1)
            return carry

        lax.fori_loop(0, _NCH // 2, pair_body, 0)

        def pack_body(sb, carry):
            for i in range(_K):
                obuf[i, pl.ds(sb * _L, _L)] = top[sb * _K + i]
            return carry

        lax.fori_loop(0, _SB, pack_body, 0)
        pltpu.sync_copy(obuf, out_hbm.at[b, :, pl.ds(d0, _FB)])
        return carry

    lax.fori_loop(0, _GPW, group_body, 0)


_kmax = pl.kernel(
    _kmax_body,
    out_type=jax.ShapeDtypeStruct((_B, _K, _D), jnp.float32),
    mesh=plsc.VectorSubcoreMesh(
        core_axis_name="c", subcore_axis_name="s",
        num_cores=_NC, num_subcores=_NS,
    ),
    compiler_params=pltpu.CompilerParams(needs_layout_passes=False),
    scratch_types=[
        pltpu.VMEM((_CH, _FB), jnp.float32),
        pltpu.VMEM((_CH, _FB), jnp.float32),
        pltpu.VMEM((_SB * _PCAP * _L,), jnp.float32),
        pltpu.VMEM((_SB * _K, _L), jnp.float32),
        pltpu.VMEM((_K, _FB), jnp.float32),
        pltpu.VMEM((_SB * _NREG, _L), jnp.int32),
        pltpu.SemaphoreType.DMA,
        pltpu.SemaphoreType.DMA,
    ],
)


@jax.jit
def kernel(x):
    return _kmax(x)


# speedup vs baseline: 1.0467x; 1.0467x over previous
import jax
import jax.numpy as jnp
from jax import lax
from jax.experimental import pallas as pl
from jax.experimental.pallas import tpu as pltpu
from jax.experimental.pallas import tpu_sc as plsc

_B, _S, _D = 4, 8192, 2048
_K = 64
_L = 16
_PCAP = 64
_CHECK = 32
_NREG = 4
_RC = _PCAP // _NREG
_FB = 128
_SB = _FB // _L
_CH = 128
_NCH = _S // _CH
_NWIN = _CH // _CHECK
_NC, _NS = 2, 16
_NW = _NC * _NS
_DBLK = _D // _FB
_GROUPS = _B * _DBLK
_GPW = _GROUPS // _NW
_NEG = float("-inf")


def _ce(vals, i, j):
    a, b = vals[i], vals[j]
    vals[i] = jnp.maximum(a, b)
    vals[j] = jnp.minimum(a, b)


def _sort_desc(vals):
    n = len(vals)
    k = 2
    while k <= n:
        j = k // 2
        while j >= 1:
            for i in range(n):
                l = i ^ j
                if l > i:
                    if (i & k) == 0:
                        _ce(vals, i, l)
                    else:
                        _ce(vals, l, i)
            j //= 2
        k *= 2


def _clean_desc(vals):
    n = len(vals)
    j = n // 2
    while j >= 1:
        for i in range(n):
            l = i ^ j
            if l > i:
                _ce(vals, i, l)
        j //= 2
    return vals


def _kmax_body(x_hbm, out_hbm, buf0, buf1, pend, top, obuf, cbuf, sem0, sem1):
    cid = lax.axis_index("c")
    sid = lax.axis_index("s")
    wid = sid * _NC + cid
    lanes = lax.iota(jnp.int32, _L)
    neg = jnp.full((_L,), _NEG, jnp.float32)

    def group_body(g, carry):
        gid = wid * _GPW + g
        b = gid // _DBLK
        d0 = (gid % _DBLK) * _FB

        def init_body(i, carry):
            top[i] = neg
            pend[pl.ds(i * _L, _L)] = neg
            return carry

        lax.fori_loop(0, _SB * _K, init_body, 0)

        def cinit_body(sb, carry):
            pb = lanes + sb * (_PCAP * _L)
            for r in range(_NREG):
                cbuf[sb * _NREG + r] = pb + r * (_RC * _L)
            return carry

        lax.fori_loop(0, _SB, cinit_body, 0)

        def dma(ci, bufp, semp):
            return pltpu.make_async_copy(
                x_hbm.at[b, pl.ds(ci * _CH, _CH), pl.ds(d0, _FB)], bufp, semp
            )

        dma(0, buf0, sem0).start()
        dma(1, buf1, sem1).start()

        def chunk_steps(ci, buf, semp):
            dma(ci, buf, semp).wait()

            def sb_body(sb, carry):
                base = sb * _K
                pbase = sb * (_PCAP * _L)
                rbases = [
                    lanes + (pbase + r * (_RC * _L)) for r in range(_NREG)
                ]

                def flush():
                    p = [pend[pl.ds(pbase + i * _L, _L)] for i in range(_PCAP)]
                    _sort_desc(p)
                    r = [top[base + i] for i in range(_K)]
                    new = _clean_desc(
                        [jnp.maximum(r[i], p[_K - 1 - i]) for i in range(_K)]
                    )
                    for i in range(_K):
                        top[base + i] = new[i]
                    for i in range(_PCAP):
                        pend[pl.ds(pbase + i * _L, _L)] = neg

                def win_body(w, ct):
                    ptrs0, t = ct

                    @plsc.parallel_loop(
                        w * _CHECK, (w + 1) * _CHECK, step=_NREG,
                        unroll=2, carry=tuple(ptrs0),
                    )
                    def rows(ri, ps):
                        out = []
                        for r in range(_NREG):
                            v = buf[ri + r, pl.ds(sb * _L, _L)]
                            m = v > t
                            plsc.store_scatter(pend, [ps[r]], v, mask=m)
                            out.append(ps[r] + jnp.where(m, _L, 0))
                        return tuple(out)

                    ptrs = list(rows)
                    cm = ptrs[0] - rbases[0]
                    for r in range(1, _NREG):
                        cm = jnp.maximum(cm, ptrs[r] - rbases[r])
                    last = (ci == _NCH - 1) & (w == _NWIN - 1)
                    do_flush = (jnp.max(cm) > (_RC * _L) // 2) | last
                    pl.when(do_flush)(flush)
                    t = top[base + _K - 1]
                    ptrs = [
                        jnp.where(do_flush, rbases[r], ptrs[r])
                        for r in range(_NREG)
                    ]
                    return tuple(ptrs), t

                p0 = tuple(cbuf[sb * _NREG + r] for r in range(_NREG))
                t0 = top[base + _K - 1]
                p1, _t1 = lax.fori_loop(0, _NWIN, win_body, (p0, t0))
                for r in range(_NREG):
                    cbuf[sb * _NREG + r] = p1[r]
                return carry

            lax.fori_loop(0, _SB, sb_body, 0)

            def refill():
                dma(ci + 2, buf, semp).start()

            pl.when(ci + 2 < _NCH)(refill)

        def pair_body(cp, carry):
            chunk_steps(cp * 2, buf0, sem0)
            chunk_steps(cp * 2 + 1, buf1, sem1)
            return carry

        lax.fori_loop(0, _NCH // 2, pair_body, 0)

        def pack_body(sb, carry):
            for i in range(_K):
                obuf[i, pl.ds(sb * _L, _L)] = top[sb * _K + i]
            return carry

        lax.fori_loop(0, _SB, pack_body, 0)
        pltpu.sync_copy(obuf, out_hbm.at[b, :, pl.ds(d0, _FB)])
        return carry

    lax.fori_loop(0, _GPW, group_body, 0)


_kmax = pl.kernel(
    _kmax_body,
    out_type=jax.ShapeDtypeStruct((_B, _K, _D), jnp.float32),
    mesh=plsc.VectorSubcoreMesh(
        core_axis_name="c", subcore_axis_name="s",
        num_cores=_NC, num_subcores=_NS,
    ),
    compiler_params=pltpu.CompilerParams(needs_layout_passes=False),
    scratch_types=[
        pltpu.VMEM((_CH, _FB), jnp.float32),
        pltpu.VMEM((_CH, _FB), jnp.float32),
        pltpu.VMEM((_SB * _PCAP * _L,), jnp.float32),
        pltpu.VMEM((_SB * _K, _L), jnp.float32),
        pltpu.VMEM((_K, _FB), jnp.float32),
        pltpu.VMEM((_SB * _NREG, _L), jnp.int32),
        pltpu.SemaphoreType.DMA,
        pltpu.SemaphoreType.DMA,
    ],
)


@jax.jit
def kernel(x):
    return _kmax(x)


# speedup vs baseline: 1.0806x; 1.0324x over previous
import jax
import jax.numpy as jnp
from jax import lax
from jax.experimental import pallas as pl
from jax.experimental.pallas import tpu as pltpu
from jax.experimental.pallas import tpu_sc as plsc

_B, _S, _D = 4, 8192, 2048
_K = 64
_L = 16
_PCAP = 64
_CHECK = 32
_NREG = 4
_RC = _PCAP // _NREG
_FB = 128
_SB = _FB // _L
_CH = 128
_NCH = _S // _CH
_NWIN = _CH // _CHECK
_NC, _NS = 2, 16
_NW = _NC * _NS
_DBLK = _D // _FB
_GROUPS = _B * _DBLK
_GPW = _GROUPS // _NW
_NEG = float("-inf")


def _ce(vals, i, j):
    a, b = vals[i], vals[j]
    vals[i] = jnp.maximum(a, b)
    vals[j] = jnp.minimum(a, b)


def _sort_desc(vals):
    n = len(vals)
    k = 2
    while k <= n:
        j = k // 2
        while j >= 1:
            for i in range(n):
                l = i ^ j
                if l > i:
                    if (i & k) == 0:
                        _ce(vals, i, l)
                    else:
                        _ce(vals, l, i)
            j //= 2
        k *= 2


def _clean_desc(vals):
    n = len(vals)
    j = n // 2
    while j >= 1:
        for i in range(n):
            l = i ^ j
            if l > i:
                _ce(vals, i, l)
        j //= 2
    return vals


def _kmax_body(x_hbm, out_hbm, buf0, buf1, pend, top, obuf, cbuf, sem0, sem1):
    cid = lax.axis_index("c")
    sid = lax.axis_index("s")
    wid = sid * _NC + cid
    lanes = lax.iota(jnp.int32, _L)
    neg = jnp.full((_L,), _NEG, jnp.float32)

    def group_body(g, carry):
        gid = wid * _GPW + g
        b = gid // _DBLK
        d0 = (gid % _DBLK) * _FB

        def init_body(i, carry):
            top[i] = neg
            pend[pl.ds(i * _L, _L)] = neg
            return carry

        lax.fori_loop(0, _SB * _K, init_body, 0)

        def cinit_body(sb, carry):
            pb = lanes + sb * (_PCAP * _L)
            for r in range(_NREG):
                cbuf[sb * _NREG + r] = pb + r * (_RC * _L)
            return carry

        lax.fori_loop(0, _SB, cinit_body, 0)

        def dma(ci, bufp, semp):
            return pltpu.make_async_copy(
                x_hbm.at[b, pl.ds(ci * _CH, _CH), pl.ds(d0, _FB)], bufp, semp
            )

        dma(0, buf0, sem0).start()
        dma(1, buf1, sem1).start()

        def chunk_steps(ci, buf, semp):
            dma(ci, buf, semp).wait()

            def sb_body(sb, carry):
                base = sb * _K
                pbase = sb * (_PCAP * _L)
                rbases = [
                    lanes + (pbase + r * (_RC * _L)) for r in range(_NREG)
                ]

                def flush():
                    p = [pend[pl.ds(pbase + i * _L, _L)] for i in range(_PCAP)]
                    _sort_desc(p)
                    r = [top[base + i] for i in range(_K)]
                    new = _clean_desc(
                        [jnp.maximum(r[i], p[_K - 1 - i]) for i in range(_K)]
                    )
                    for i in range(_K):
                        top[base + i] = new[i]
                    for i in range(_PCAP):
                        pend[pl.ds(pbase + i * _L, _L)] = neg

                def win_body(w, ct):
                    ptrs0, t = ct

                    @plsc.parallel_loop(
                        w * _CHECK, (w + 1) * _CHECK, step=_NREG,
                        unroll=4, carry=tuple(ptrs0),
                    )
                    def rows(ri, ps):
                        out = []
                        for r in range(_NREG):
                            v = buf[ri + r, pl.ds(sb * _L, _L)]
                            m = v > t
                            plsc.store_scatter(pend, [ps[r]], v, mask=m)
                            out.append(ps[r] + jnp.where(m, _L, 0))
                        return tuple(out)

                    ptrs = list(rows)
                    cm = ptrs[0] - rbases[0]
                    for r in range(1, _NREG):
                        cm = jnp.maximum(cm, ptrs[r] - rbases[r])
                    last = (ci == _NCH - 1) & (w == _NWIN - 1)
                    do_flush = (jnp.max(cm) > (_RC * _L) // 2) | last
                    pl.when(do_flush)(flush)
                    t = top[base + _K - 1]
                    ptrs = [
                        jnp.where(do_flush, rbases[r], ptrs[r])
                        for r in range(_NREG)
                    ]
                    return tuple(ptrs), t

                p0 = tuple(cbuf[sb * _NREG + r] for r in range(_NREG))
                t0 = top[base + _K - 1]
                p1, _t1 = lax.fori_loop(0, _NWIN, win_body, (p0, t0))
                for r in range(_NREG):
                    cbuf[sb * _NREG + r] = p1[r]
                return carry

            lax.fori_loop(0, _SB, sb_body, 0)

            def refill():
                dma(ci + 2, buf, semp).start()

            pl.when(ci + 2 < _NCH)(refill)

        def pair_body(cp, carry):
            chunk_steps(cp * 2, buf0, sem0)
            chunk_steps(cp * 2 + 1, buf1, sem1)
            return carry

        lax.fori_loop(0, _NCH // 2, pair_body, 0)

        def pack_body(sb, carry):
            for i in range(_K):
                obuf[i, pl.ds(sb * _L, _L)] = top[sb * _K + i]
            return carry

        lax.fori_loop(0, _SB, pack_body, 0)
        pltpu.sync_copy(obuf, out_hbm.at[b, :, pl.ds(d0, _FB)])
        return carry

    lax.fori_loop(0, _GPW, group_body, 0)


_kmax = pl.kernel(
    _kmax_body,
    out_type=jax.ShapeDtypeStruct((_B, _K, _D), jnp.float32),
    mesh=plsc.VectorSubcoreMesh(
        core_axis_name="c", subcore_axis_name="s",
        num_cores=_NC, num_subcores=_NS,
    ),
    compiler_params=pltpu.CompilerParams(needs_layout_passes=False),
    scratch_types=[
        pltpu.VMEM((_CH, _FB), jnp.float32),
        pltpu.VMEM((_CH, _FB), jnp.float32),
        pltpu.VMEM((_SB * _PCAP * _L,), jnp.float32),
        pltpu.VMEM((_SB * _K, _L), jnp.float32),
        pltpu.VMEM((_K, _FB), jnp.float32),
        pltpu.VMEM((_SB * _NREG, _L), jnp.int32),
        pltpu.SemaphoreType.DMA,
        pltpu.SemaphoreType.DMA,
    ],
)


@jax.jit
def kernel(x):
    return _kmax(x)
